# baseline (speedup 1.0000x reference)
import functools

import jax
import jax.numpy as jnp
from jax import lax
from jax.experimental import pallas as pl
from jax.experimental.pallas import tpu as pltpu
from jax.experimental.pallas import tpu_sc as plsc

_BK = 40000


def _argmax_body(q_ref, db_ref, idx_out_ref, bv_ref, bi_ref, *, bk, k_total,
                 nsteps):
    i = pl.program_id(0)

    @pl.when(i == 0)
    def _init():
        bv_ref[...] = jnp.full_like(bv_ref, -jnp.inf)
        bi_ref[...] = jnp.zeros_like(bi_ref)

    sim = lax.dot_general(
        q_ref[...], db_ref[...],
        (((1,), (1,)), ((), ())),
        preferred_element_type=jnp.float32,
    )
    m = jnp.max(sim, axis=1, keepdims=True)
    bv = bv_ref[...]
    better = m > bv

    @pl.when(jnp.any(better))
    def _update():
        iota = lax.broadcasted_iota(jnp.int32, sim.shape, 1)
        li = jnp.min(jnp.where(sim == m, iota, k_total), axis=1, keepdims=True)
        bi_ref[...] = jnp.where(better, i * bk + li, bi_ref[...])
        bv_ref[...] = jnp.where(better, m, bv)

    @pl.when(i == nsteps - 1)
    def _emit():
        idx_out_ref[...] = bi_ref[...]


def _tc_argmax(q, db, bk, interpret=False):
    k_total, d = db.shape
    nq = q.shape[0]
    nsteps = k_total // bk
    return pl.pallas_call(
        functools.partial(_argmax_body, bk=bk, k_total=k_total, nsteps=nsteps),
        grid=(nsteps,),
        in_specs=[
            pl.BlockSpec((nq, d), lambda i: (0, 0)),
            pl.BlockSpec((bk, d), lambda i: (i, 0)),
        ],
        out_specs=pl.BlockSpec((nq, 1), lambda i: (0, 0)),
        out_shape=jax.ShapeDtypeStruct((nq, 1), jnp.int32),
        scratch_shapes=[
            pltpu.VMEM((nq, 1), jnp.float32),
            pltpu.VMEM((nq, 1), jnp.int32),
        ],
        compiler_params=pltpu.CompilerParams(
            dimension_semantics=("arbitrary",),
        ),
        interpret=interpret,
    )(q, db)


def _sc_gather(idx, cat, shp):
    n = idx.shape[0]
    mesh = plsc.VectorSubcoreMesh(core_axis_name="c", subcore_axis_name="s")

    @functools.partial(
        pl.kernel,
        mesh=mesh,
        out_type=[
            jax.ShapeDtypeStruct((n,), jnp.int32),
            jax.ShapeDtypeStruct((n,), jnp.int32),
        ],
        scratch_types=[
            pltpu.VMEM((n,), jnp.int32),
            pltpu.VMEM((n,), jnp.int32),
            pltpu.SemaphoreType.DMA,
        ],
    )
    def gather_kernel(idx_hbm, cat_hbm, shp_hbm, cat_out, shp_out, idx_v,
                      val_v, sem):
        c = lax.axis_index("c")
        s = lax.axis_index("s")
        wid = s * 2 + c

        @pl.when(wid == 0)
        def _cat():
            pltpu.sync_copy(idx_hbm, idx_v)
            pltpu.async_copy(cat_hbm.at[idx_v], val_v, sem).wait()
            pltpu.sync_copy(val_v, cat_out)

        @pl.when(wid == 1)
        def _shp():
            pltpu.sync_copy(idx_hbm, idx_v)
            pltpu.async_copy(shp_hbm.at[idx_v], val_v, sem).wait()
            pltpu.sync_copy(val_v, shp_out)

    return gather_kernel(idx, cat, shp)


def kernel(shape_embedding, db_embedding, category_idx, shape_idx):
    idx = _tc_argmax(shape_embedding, db_embedding, _BK).reshape(-1)
    cat, shp = _sc_gather(idx, category_idx, shape_idx)
    return cat, shp
